# Initial kernel scaffold; baseline (speedup 1.0000x reference)
import jax
import jax.numpy as jnp
from jax.experimental import pallas as pl


def kernel(x, W):
    raise NotImplementedError("write your pallas kernel here")


# speedup vs baseline: 1.1130x; 1.1130x over previous
import jax
import jax.numpy as jnp
from jax.experimental import pallas as pl

_TOPK = 8
_BLK = 512


def _gate_body(x_ref, w_ref, out_ref):
    x = x_ref[...]
    w = w_ref[...]
    n_experts = w.shape[0]
    logits = jax.lax.dot_general(
        x, w, (((1,), (1,)), ((), ())), preferred_element_type=jnp.float32
    )
    m = jnp.max(logits, axis=-1, keepdims=True)
    e = jnp.exp(logits - m)
    p = e / jnp.sum(e, axis=-1, keepdims=True)
    iota = jax.lax.broadcasted_iota(jnp.int32, p.shape, 1)
    cols = []
    for _ in range(_TOPK):
        mk = jnp.max(p, axis=-1, keepdims=True)
        cand = jnp.where(p == mk, iota, n_experts)
        idx = jnp.min(cand, axis=-1, keepdims=True)
        cols.append(idx)
        p = jnp.where(iota == idx, -jnp.inf, p)
    out_ref[...] = jnp.concatenate(cols, axis=1)


def kernel(x, W):
    rows, d = x.shape
    n_experts = W.shape[0]
    return pl.pallas_call(
        _gate_body,
        grid=(rows // _BLK,),
        in_specs=[
            pl.BlockSpec((_BLK, d), lambda i: (i, 0)),
            pl.BlockSpec((n_experts, d), lambda i: (0, 0)),
        ],
        out_specs=pl.BlockSpec((_BLK, _TOPK), lambda i: (i, 0)),
        out_shape=jax.ShapeDtypeStruct((rows, _TOPK), jnp.int32),
    )(x, W)


# speedup vs baseline: 1.1748x; 1.0555x over previous
import functools

import jax
import jax.numpy as jnp
from jax import lax
from jax.experimental import pallas as pl
from jax.experimental.pallas import tpu as pltpu
from jax.experimental.pallas import tpu_sc as plsc

_TOPK = 8
_L = 16
_NW = 32


def _tc_logits_body(x_ref, w_ref, lt_ref):
    lt_ref[...] = jax.lax.dot_general(
        w_ref[...], x_ref[...], (((1,), (1,)), ((), ())),
        preferred_element_type=jnp.float32,
    )


def _tc_logits(x, W, blk):
    rows, d = x.shape
    n_experts = W.shape[0]
    return pl.pallas_call(
        _tc_logits_body,
        grid=(rows // blk,),
        in_specs=[
            pl.BlockSpec((blk, d), lambda i: (i, 0)),
            pl.BlockSpec((n_experts, d), lambda i: (0, 0)),
        ],
        out_specs=pl.BlockSpec((n_experts, blk), lambda i: (0, i)),
        out_shape=jax.ShapeDtypeStruct((n_experts, rows), jnp.float32),
    )(x, W)


def _sc_topk_body(rpw, n_experts, lt_hbm, out_hbm, tile_v, out_v):
    cid = lax.axis_index("c")
    sid = lax.axis_index("s")
    wid = sid * 2 + cid
    base = wid * rpw
    pltpu.sync_copy(lt_hbm.at[:, pl.ds(base, rpw)], tile_v)

    lanes = lax.broadcasted_iota(jnp.int32, (_L,), 0)
    neg_inf = jnp.full((_L,), -jnp.inf, jnp.float32)
    zeros_i = jnp.zeros((_L,), jnp.int32)

    def group(g, carry):
        tv = [neg_inf] * _TOPK
        ti = [zeros_i] * _TOPK
        col = g * _L
        for e in range(n_experts):
            v = tile_v[e, pl.ds(col, _L)]
            vi = jnp.full((_L,), e, jnp.int32)
            for j in range(_TOPK):
                c = v > tv[j]
                ntv = jnp.where(c, v, tv[j])
                nti = jnp.where(c, vi, ti[j])
                v = jnp.where(c, tv[j], v)
                vi = jnp.where(c, ti[j], vi)
                tv[j] = ntv
                ti[j] = nti
        flat0 = (col + lanes) * _TOPK
        for j in range(_TOPK):
            plsc.store_scatter(out_v, [flat0 + j], ti[j])
        return carry

    lax.fori_loop(0, rpw // _L, group, 0)
    pltpu.sync_copy(out_v, out_hbm.at[pl.ds(base * _TOPK, rpw * _TOPK)])


def _sc_topk(lt):
    n_experts, rows = lt.shape
    rpw = rows // _NW
    mesh = plsc.VectorSubcoreMesh(core_axis_name="c", subcore_axis_name="s")
    f = pl.kernel(
        functools.partial(_sc_topk_body, rpw, n_experts),
        out_type=jax.ShapeDtypeStruct((rows * _TOPK,), jnp.int32),
        mesh=mesh,
        compiler_params=pltpu.CompilerParams(needs_layout_passes=False),
        scratch_types=[
            pltpu.VMEM((n_experts, rpw), jnp.float32),
            pltpu.VMEM((rpw * _TOPK,), jnp.int32),
        ],
    )
    return f(lt).reshape(rows, _TOPK)


def kernel(x, W):
    lt = _tc_logits(x, W, blk=512)
    return _sc_topk(lt)


# speedup vs baseline: 1.3583x; 1.1562x over previous
import functools

import jax
import jax.numpy as jnp
from jax import lax
from jax.experimental import pallas as pl
from jax.experimental.pallas import tpu as pltpu
from jax.experimental.pallas import tpu_sc as plsc

_TOPK = 8
_L = 16
_NW = 32


def _tc_logits_body(x_ref, w_ref, lt_ref):
    lt_ref[...] = jax.lax.dot_general(
        w_ref[...], x_ref[...], (((1,), (1,)), ((), ())),
        preferred_element_type=jnp.float32,
    )


def _tc_logits(x, W, blk, row0, nrows):
    d = x.shape[1]
    n_experts = W.shape[0]
    blk0 = row0 // blk
    return pl.pallas_call(
        _tc_logits_body,
        grid=(nrows // blk,),
        in_specs=[
            pl.BlockSpec((blk, d), lambda i: (blk0 + i, 0)),
            pl.BlockSpec((n_experts, d), lambda i: (0, 0)),
        ],
        out_specs=pl.BlockSpec((n_experts, blk), lambda i: (0, i)),
        out_shape=jax.ShapeDtypeStruct((n_experts, nrows), jnp.float32),
    )(x, W)


def _sc_topk_body(rpw, n_experts, lt_hbm, out_hbm, tile_v, out_v):
    cid = lax.axis_index("c")
    sid = lax.axis_index("s")
    wid = sid * 2 + cid
    base = wid * rpw
    pltpu.sync_copy(lt_hbm.at[:, pl.ds(base, rpw)], tile_v)

    lanes = lax.broadcasted_iota(jnp.int32, (_L,), 0)
    neg_inf = jnp.full((_L,), -jnp.inf, jnp.float32)
    zeros_i = jnp.zeros((_L,), jnp.int32)

    def group(g, carry):
        tv = [neg_inf] * _TOPK
        ti = [zeros_i] * _TOPK
        col = g * _L
        for e in range(n_experts):
            v = tile_v[e, pl.ds(col, _L)]
            vi = jnp.full((_L,), e, jnp.int32)
            for j in range(_TOPK):
                c = v > tv[j]
                ntv = jnp.where(c, v, tv[j])
                nti = jnp.where(c, vi, ti[j])
                v = jnp.where(c, tv[j], v)
                vi = jnp.where(c, ti[j], vi)
                tv[j] = ntv
                ti[j] = nti
        flat0 = (col + lanes) * _TOPK
        for j in range(_TOPK):
            plsc.store_scatter(out_v, [flat0 + j], ti[j])
        return carry

    lax.fori_loop(0, rpw // _L, group, 0)
    pltpu.sync_copy(out_v, out_hbm.at[pl.ds(base * _TOPK, rpw * _TOPK)])


def _sc_topk(lt):
    n_experts, rows = lt.shape
    rpw = rows // _NW
    mesh = plsc.VectorSubcoreMesh(core_axis_name="c", subcore_axis_name="s")
    f = pl.kernel(
        functools.partial(_sc_topk_body, rpw, n_experts),
        out_type=jax.ShapeDtypeStruct((rows * _TOPK,), jnp.int32),
        mesh=mesh,
        compiler_params=pltpu.CompilerParams(needs_layout_passes=False),
        scratch_types=[
            pltpu.VMEM((n_experts, rpw), jnp.float32),
            pltpu.VMEM((rpw * _TOPK,), jnp.int32),
        ],
    )
    return f(lt).reshape(rows, _TOPK)


def kernel(x, W):
    rows = x.shape[0]
    n_chunks = 4
    crows = rows // n_chunks
    outs = []
    for c in range(n_chunks):
        lt = _tc_logits(x, W, 512, c * crows, crows)
        outs.append(_sc_topk(lt))
    return jnp.concatenate(outs, axis=0)


# speedup vs baseline: 1.4301x; 1.0529x over previous
import functools

import jax
import jax.numpy as jnp
from jax import lax
from jax.experimental import pallas as pl
from jax.experimental.pallas import tpu as pltpu
from jax.experimental.pallas import tpu_sc as plsc

_TOPK = 8
_L = 16
_NW = 32


def _tc_logits_body(x_ref, w_ref, lt_ref):
    lt_ref[...] = jax.lax.dot_general(
        w_ref[...], x_ref[...], (((1,), (1,)), ((), ())),
        preferred_element_type=jnp.float32,
    )


def _tc_logits(x, W, blk, row0, nrows):
    d = x.shape[1]
    n_experts = W.shape[0]
    blk0 = row0 // blk
    return pl.pallas_call(
        _tc_logits_body,
        grid=(nrows // blk,),
        in_specs=[
            pl.BlockSpec((blk, d), lambda i: (blk0 + i, 0)),
            pl.BlockSpec((n_experts, d), lambda i: (0, 0)),
        ],
        out_specs=pl.BlockSpec((n_experts, blk), lambda i: (0, i)),
        out_shape=jax.ShapeDtypeStruct((n_experts, nrows), jnp.float32),
    )(x, W)


def _sc_topk_body(rpw, n_experts, lt_hbm, out_hbm, tile_v, out_v):
    cid = lax.axis_index("c")
    sid = lax.axis_index("s")
    wid = sid * 2 + cid
    base = wid * rpw
    pltpu.sync_copy(lt_hbm.at[:, pl.ds(base, rpw)], tile_v)

    lanes = lax.broadcasted_iota(jnp.int32, (_L,), 0)
    neg_inf = jnp.full((_L,), -jnp.inf, jnp.float32)
    zeros_i = jnp.zeros((_L,), jnp.int32)

    def group(g, carry):
        tv = [neg_inf] * _TOPK
        ti = [zeros_i] * _TOPK
        col = g * _L
        for e in range(n_experts):
            v = tile_v[e, pl.ds(col, _L)]
            vi = jnp.full((_L,), e, jnp.int32)
            for j in range(_TOPK):
                c = v > tv[j]
                ntv = jnp.where(c, v, tv[j])
                nti = jnp.where(c, vi, ti[j])
                v = jnp.where(c, tv[j], v)
                vi = jnp.where(c, ti[j], vi)
                tv[j] = ntv
                ti[j] = nti
        flat0 = (col + lanes) * _TOPK
        for j in range(_TOPK):
            plsc.store_scatter(out_v, [flat0 + j], ti[j])
        return carry

    lax.fori_loop(0, rpw // _L, group, 0)
    pltpu.sync_copy(out_v, out_hbm.at[pl.ds(base * _TOPK, rpw * _TOPK)])


def _sc_topk(lt):
    n_experts, rows = lt.shape
    rpw = rows // _NW
    mesh = plsc.VectorSubcoreMesh(core_axis_name="c", subcore_axis_name="s")
    f = pl.kernel(
        functools.partial(_sc_topk_body, rpw, n_experts),
        out_type=jax.ShapeDtypeStruct((rows * _TOPK,), jnp.int32),
        mesh=mesh,
        compiler_params=pltpu.CompilerParams(needs_layout_passes=False),
        scratch_types=[
            pltpu.VMEM((n_experts, rpw), jnp.float32),
            pltpu.VMEM((rpw * _TOPK,), jnp.int32),
        ],
    )
    return f(lt).reshape(rows, _TOPK)


def kernel(x, W):
    rows = x.shape[0]
    n_chunks = 4
    crows = rows // n_chunks
    outs = []
    for c in range(n_chunks):
        lt = _tc_logits(x, W, 1024, c * crows, crows)
        outs.append(_sc_topk(lt))
    return jnp.concatenate(outs, axis=0)
